# Initial kernel scaffold; baseline (speedup 1.0000x reference)
import jax
import jax.numpy as jnp
from jax.experimental import pallas as pl


def kernel(spatial_pos, edge_input, spatial_w, edge_w, edge_dis_w, graph_token):
    raise NotImplementedError("write your pallas kernel here")


# speedup vs baseline: 5.8741x; 5.8741x over previous
import functools

import jax
import jax.numpy as jnp
from jax import lax
from jax.experimental import pallas as pl
from jax.experimental.pallas import tpu as pltpu
from jax.experimental.pallas import tpu_sc as plsc

H = 32
NUM_SPATIAL = 512
EW_PAD = 1032
D = 4
T_ROWS = NUM_SPATIAL + D * EW_PAD
B, N = 16, 128
POS = B * N * N
ROWS = POS // N
NW = 32
ROWS_PER_W = ROWS // NW


def _prep_body(sw_ref, ew_ref, w4_ref, t_ref):
    sw_rows = lax.broadcasted_iota(jnp.int32, (NUM_SPATIAL, H), 0)
    sw0 = jnp.where(sw_rows == 0, 0.0, sw_ref[...])
    ew_rows = lax.broadcasted_iota(jnp.int32, (EW_PAD, H), 0)
    ew0 = jnp.where(ew_rows == 0, 0.0, ew_ref[...])
    t_ref[0:NUM_SPATIAL, :] = sw0
    for d in range(D):
        em = jnp.dot(ew0, w4_ref[d], preferred_element_type=jnp.float32)
        t_ref[NUM_SPATIAL + d * EW_PAD:NUM_SPATIAL + (d + 1) * EW_PAD, :] = em


def _build_table(spatial_w, ew_pad, w4):
    return pl.pallas_call(
        _prep_body,
        out_shape=jax.ShapeDtypeStruct((T_ROWS, H), jnp.float32),
    )(spatial_w, ew_pad, w4)


def _sc_body(t_hbm, sp_hbm, ei_hbm, out_hbm,
             spi_v, ei_v, srow_v, erow_v, inv_v, out_v, sem):
    wid = lax.axis_index("c") * 16 + lax.axis_index("s")
    r0 = wid * ROWS_PER_W

    d_offs = NUM_SPATIAL + (lax.iota(jnp.int32, 16) % 4) * EW_PAD

    def row_step(k, carry):
        r = r0 + k
        pltpu.sync_copy(sp_hbm.at[r], spi_v)
        pltpu.sync_copy(ei_hbm.at[pl.ds(r * 4, 4)], ei_v)
        for j in range(4):
            for t in range(8):
                sl = pl.ds(t * 16, 16)
                ei_v[j, sl] = ei_v[j, sl] + d_offs
        for t in range(8):
            sl = pl.ds(t * 16, 16)
            sp = spi_v[sl]
            sp = jnp.where(sp == 0, 1, sp)
            sp = jnp.where(sp > 1, sp - 1, sp)
            inv_v[sl] = 1.0 / (sp.astype(jnp.float32) + 1e-9)
        cps = [pltpu.async_copy(t_hbm.at[spi_v], srow_v, sem)]
        for j in range(4):
            cps.append(pltpu.async_copy(
                t_hbm.at[ei_v.at[j]], erow_v.at[pl.ds(j * N, N)], sem))
        for cp in cps:
            cp.wait()

        for g in range(N // 16):
            invg = inv_v[pl.ds(g * 16, 16)]
            for l in range(16):
                p = g * 16 + l
                iv = invg[l]
                for h2 in range(2):
                    sl = pl.ds(h2 * 16, 16)
                    acc = (erow_v[4 * p, sl] + erow_v[4 * p + 1, sl]) + (
                        erow_v[4 * p + 2, sl] + erow_v[4 * p + 3, sl])
                    out_v[p, sl] = srow_v[p, sl] + acc * iv
        pltpu.sync_copy(out_v, out_hbm.at[r])
        return carry

    lax.fori_loop(0, ROWS_PER_W, row_step, 0)


def _sc_gather_combine(table, sp_flat, ei_flat):
    mesh = plsc.VectorSubcoreMesh(core_axis_name="c", subcore_axis_name="s")
    f = functools.partial(
        pl.kernel,
        mesh=mesh,
        compiler_params=pltpu.CompilerParams(use_tc_tiling_on_sc=False),
        out_type=jax.ShapeDtypeStruct((ROWS, N, H), jnp.float32),
        scratch_types=[
            pltpu.VMEM((N,), jnp.int32),
            pltpu.VMEM((4, N), jnp.int32),
            pltpu.VMEM((N, H), jnp.float32),
            pltpu.VMEM((4 * N, H), jnp.float32),
            pltpu.VMEM((N,), jnp.float32),
            pltpu.VMEM((N, H), jnp.float32),
            pltpu.SemaphoreType.DMA,
        ],
    )(_sc_body)
    return f(table, sp_flat, ei_flat)


def kernel(spatial_pos, edge_input, spatial_w, edge_w, edge_dis_w, graph_token):
    ew_pad = jnp.pad(edge_w, ((0, EW_PAD - edge_w.shape[0]), (0, 0)))
    w4 = edge_dis_w.reshape(-1, H, H)[:D]
    table = _build_table(spatial_w, ew_pad, w4)

    sp_flat = spatial_pos.astype(jnp.int32).reshape(ROWS, N)
    ei_flat = edge_input.astype(jnp.int32).reshape(ROWS * 4, N)

    core = _sc_gather_combine(table, sp_flat, ei_flat)

    core = core.reshape(B, N, N, H)
    bias = jnp.transpose(core, (0, 3, 1, 2))
    bias = jnp.pad(bias, ((0, 0), (0, 0), (1, 0), (1, 0)))
    t = graph_token.reshape(1, H, 1)
    bias = bias.at[:, :, 1:, 0].add(t)
    bias = bias.at[:, :, 0, 1:].add(t)
    return bias


# speedup vs baseline: 7.2701x; 1.2377x over previous
import functools

import jax
import jax.numpy as jnp
from jax import lax
from jax.experimental import pallas as pl
from jax.experimental.pallas import tpu as pltpu
from jax.experimental.pallas import tpu_sc as plsc

H = 32
NUM_SPATIAL = 512
EW_PAD = 1032
D = 4
T_ROWS = NUM_SPATIAL + D * EW_PAD
B, N = 16, 128
POS = B * N * N
ROWS = POS // N
NW = 32
ROWS_PER_W = ROWS // NW


def _prep_body(sw_ref, ew_ref, w4_ref, t_ref):
    sw_rows = lax.broadcasted_iota(jnp.int32, (NUM_SPATIAL, H), 0)
    sw0 = jnp.where(sw_rows == 0, 0.0, sw_ref[...])
    ew_rows = lax.broadcasted_iota(jnp.int32, (EW_PAD, H), 0)
    ew0 = jnp.where(ew_rows == 0, 0.0, ew_ref[...])
    t_ref[0:NUM_SPATIAL, :] = sw0
    for d in range(D):
        em = jnp.dot(ew0, w4_ref[d], preferred_element_type=jnp.float32)
        t_ref[NUM_SPATIAL + d * EW_PAD:NUM_SPATIAL + (d + 1) * EW_PAD, :] = em


def _build_table(spatial_w, ew_pad, w4):
    return pl.pallas_call(
        _prep_body,
        out_shape=jax.ShapeDtypeStruct((T_ROWS, H), jnp.float32),
    )(spatial_w, ew_pad, w4)


def _sc_body(t_hbm, sp_hbm, ei_hbm, out_hbm,
             spi0, spi1, ei0, ei1, srow0, srow1, erow0, erow1,
             inv0, inv1, outv0, outv1, sg0, sg1, so0, so1):
    wid = lax.axis_index("c") * 16 + lax.axis_index("s")
    base = wid * ROWS_PER_W
    spi_v, ei_v = (spi0, spi1), (ei0, ei1)
    srow_v, erow_v = (srow0, srow1), (erow0, erow1)
    inv_v, out_v = (inv0, inv1), (outv0, outv1)
    sg, so = (sg0, sg1), (so0, so1)

    def stage(b, r):
        pltpu.sync_copy(sp_hbm.at[r], spi_v[b])
        pltpu.sync_copy(ei_hbm.at[pl.ds(r * 4, 4)], ei_v[b])
        for t in range(8):
            sl = pl.ds(t * 16, 16)
            sp = spi_v[b][sl]
            sp = jnp.where(sp == 0, 1, sp)
            sp = jnp.where(sp > 1, sp - 1, sp)
            inv_v[b][sl] = 1.0 / (sp.astype(jnp.float32) + 1e-9)
        pltpu.async_copy(t_hbm.at[spi_v[b]], srow_v[b], sg[b])
        for j in range(4):
            pltpu.async_copy(
                t_hbm.at[ei_v[b].at[j]], erow_v[b].at[pl.ds(j * N, N)], sg[b])

    def wait_gathers(b):
        pltpu.make_async_copy(t_hbm.at[spi_v[b]], srow_v[b], sg[b]).wait()
        for j in range(4):
            pltpu.make_async_copy(
                t_hbm.at[ei_v[b].at[j]], erow_v[b].at[pl.ds(j * N, N)],
                sg[b]).wait()

    def combine(b):
        def group(g, carry):
            p0 = g * 16
            invg = inv_v[b][pl.ds(p0, 16)]
            for l in range(16):
                p = p0 + l
                iv = invg[l]
                q = 4 * p
                for h2 in range(2):
                    sl = pl.ds(h2 * 16, 16)
                    acc = (erow_v[b][q, sl] + erow_v[b][q + 1, sl]) + (
                        erow_v[b][q + 2, sl] + erow_v[b][q + 3, sl])
                    out_v[b][p, sl] = srow_v[b][p, sl] + acc * iv
            return carry

        lax.fori_loop(0, N // 16, group, 0)

    def fire_out(b, r):
        pltpu.async_copy(out_v[b], out_hbm.at[r], so[b])

    def wait_out(b, r):
        pltpu.make_async_copy(out_v[b], out_hbm.at[r], so[b]).wait()

    stage(0, base)

    def body(k, carry):
        r0 = base + 2 * k
        r1 = r0 + 1
        stage(1, r1)
        wait_gathers(0)

        @pl.when(k > 0)
        def _():
            wait_out(0, r0)

        combine(0)
        fire_out(0, r0)

        @pl.when(k < ROWS_PER_W // 2 - 1)
        def _():
            stage(0, r0 + 2)

        wait_gathers(1)

        @pl.when(k > 0)
        def _():
            wait_out(1, r1)

        combine(1)
        fire_out(1, r1)
        return carry

    lax.fori_loop(0, ROWS_PER_W // 2, body, 0)
    wait_out(0, base)
    wait_out(1, base)


def _sc_gather_combine(table, sp_flat, ei_flat):
    mesh = plsc.VectorSubcoreMesh(core_axis_name="c", subcore_axis_name="s")
    f = functools.partial(
        pl.kernel,
        mesh=mesh,
        compiler_params=pltpu.CompilerParams(use_tc_tiling_on_sc=False),
        out_type=jax.ShapeDtypeStruct((ROWS, N, H), jnp.float32),
        scratch_types=(
            [pltpu.VMEM((N,), jnp.int32)] * 2
            + [pltpu.VMEM((4, N), jnp.int32)] * 2
            + [pltpu.VMEM((N, H), jnp.float32)] * 2
            + [pltpu.VMEM((4 * N, H), jnp.float32)] * 2
            + [pltpu.VMEM((N,), jnp.float32)] * 2
            + [pltpu.VMEM((N, H), jnp.float32)] * 2
            + [pltpu.SemaphoreType.DMA] * 4
        ),
    )(_sc_body)
    return f(table, sp_flat, ei_flat)


def kernel(spatial_pos, edge_input, spatial_w, edge_w, edge_dis_w, graph_token):
    ew_pad = jnp.pad(edge_w, ((0, EW_PAD - edge_w.shape[0]), (0, 0)))
    w4 = edge_dis_w.reshape(-1, H, H)[:D]
    table = _build_table(spatial_w, ew_pad, w4)

    sp_flat = spatial_pos.astype(jnp.int32).reshape(ROWS, N)
    d_offs = NUM_SPATIAL + (jnp.arange(N, dtype=jnp.int32) % 4) * EW_PAD
    ei_flat = edge_input.astype(jnp.int32).reshape(ROWS * 4, N) + d_offs[None, :]

    core = _sc_gather_combine(table, sp_flat, ei_flat)

    core = core.reshape(B, N, N, H)
    bias = jnp.transpose(core, (0, 3, 1, 2))
    bias = jnp.pad(bias, ((0, 0), (0, 0), (1, 0), (1, 0)))
    t = graph_token.reshape(1, H, 1)
    bias = bias.at[:, :, 1:, 0].add(t)
    bias = bias.at[:, :, 0, 1:].add(t)
    return bias
